# Initial kernel scaffold; baseline (speedup 1.0000x reference)
import jax
import jax.numpy as jnp
from jax.experimental import pallas as pl


def kernel(positions, pe):
    raise NotImplementedError("write your pallas kernel here")


# speedup vs baseline: 1.9819x; 1.9819x over previous
import functools

import jax
import jax.numpy as jnp
from jax import lax
from jax.experimental import pallas as pl
from jax.experimental.pallas import tpu as pltpu
from jax.experimental.pallas import tpu_sc as plsc

DIM = 64
BATCH = 16384
NUM_CORES = 2
NUM_SUBCORES = 16
NUM_WORKERS = NUM_CORES * NUM_SUBCORES
ROWS_PER_WORKER = BATCH // NUM_WORKERS
CHUNK = 128
CHUNKS_PER_WORKER = ROWS_PER_WORKER // CHUNK


def _gather_body(pos_hbm, pe_hbm, out_hbm, idx_v, rows_v, sem):
    wid = lax.axis_index("s") * NUM_CORES + lax.axis_index("c")
    base = wid * CHUNKS_PER_WORKER
    pltpu.sync_copy(pos_hbm.at[pl.ds(base, CHUNKS_PER_WORKER)], idx_v)
    copies = []
    for j in range(CHUNKS_PER_WORKER):
        copies.append(
            pltpu.async_copy(
                pe_hbm.at[idx_v.at[j]],
                rows_v.at[pl.ds(j * CHUNK, CHUNK)],
                sem,
            )
        )
    for c in copies:
        c.wait()
    pltpu.sync_copy(rows_v, out_hbm.at[pl.ds(wid * ROWS_PER_WORKER, ROWS_PER_WORKER)])


@jax.jit
def kernel(positions, pe):
    pos2d = positions.astype(jnp.int32).reshape(NUM_WORKERS * CHUNKS_PER_WORKER, CHUNK)
    mesh = plsc.VectorSubcoreMesh(core_axis_name="c", subcore_axis_name="s")
    run = functools.partial(
        pl.kernel,
        mesh=mesh,
        out_type=jax.ShapeDtypeStruct((BATCH, DIM), jnp.float32),
        scratch_types=[
            pltpu.VMEM((CHUNKS_PER_WORKER, CHUNK), jnp.int32),
            pltpu.VMEM((ROWS_PER_WORKER, DIM), jnp.float32),
            pltpu.SemaphoreType.DMA,
        ],
        compiler_params=pltpu.CompilerParams(use_tc_tiling_on_sc=False),
    )(_gather_body)
    return run(pos2d, pe)


# speedup vs baseline: 2.1993x; 1.1097x over previous
import functools

import jax
import jax.numpy as jnp
from jax import lax
from jax.experimental import pallas as pl
from jax.experimental.pallas import tpu as pltpu
from jax.experimental.pallas import tpu_sc as plsc

DIM = 64
MAX_LEN = 8192
BATCH = 16384
NUM_CORES = 2
NUM_SUBCORES = 16
NUM_WORKERS = NUM_CORES * NUM_SUBCORES
TILE_ROWS = DIM // 8
N_QUARTERS = NUM_WORKERS // TILE_ROWS
POS_PER_WORKER = BATCH // N_QUARTERS
GROUPS = POS_PER_WORKER // 16
TAB_CTILES = MAX_LEN // 128
OUT_CTILES_TOTAL = BATCH // 128
OUT_CTILES = POS_PER_WORKER // 128


def _gather_body(pos_hbm, pe4_hbm, out4_hbm, idx_v, tab_v, out_v, sem):
    wid = lax.axis_index("s") * NUM_CORES + lax.axis_index("c")
    k = wid % TILE_ROWS
    q = wid // TILE_ROWS
    cp_tab = pltpu.async_copy(pe4_hbm.at[k], tab_v, sem)
    cp_idx = pltpu.async_copy(
        pos_hbm.at[pl.ds(q * POS_PER_WORKER, POS_PER_WORKER)], idx_v, sem
    )
    cp_tab.wait()
    cp_idx.wait()

    lane_iota = jax.lax.broadcasted_iota(jnp.int32, (16,), 0)

    def group(g, carry):
        pvec = idx_v[pl.ds(g * 16, 16)]
        hi = lax.shift_right_logical(pvec, 7)
        lo = lax.bitwise_and(pvec, jnp.int32(127))
        i16 = lane_iota + g * 16
        ct = lax.shift_right_logical(i16, 7)
        lane = lax.bitwise_and(i16, jnp.int32(127))
        for d8 in range(8):
            dvec = jnp.full((16,), d8, jnp.int32)
            v = plsc.load_gather(tab_v, [hi, dvec, lo])
            plsc.store_scatter(out_v, [ct, dvec, lane], v)
        return carry

    lax.fori_loop(0, GROUPS, group, 0)
    pltpu.sync_copy(
        out_v, out4_hbm.at[k, pl.ds(q * OUT_CTILES, OUT_CTILES)]
    )


@jax.jit
def kernel(positions, pe):
    pe4 = pe.T.reshape(TILE_ROWS, 8, TAB_CTILES, 128).transpose(0, 2, 1, 3)
    mesh = plsc.VectorSubcoreMesh(core_axis_name="c", subcore_axis_name="s")
    run = functools.partial(
        pl.kernel,
        mesh=mesh,
        out_type=jax.ShapeDtypeStruct(
            (TILE_ROWS, OUT_CTILES_TOTAL, 8, 128), jnp.float32
        ),
        scratch_types=[
            pltpu.VMEM((POS_PER_WORKER,), jnp.int32),
            pltpu.VMEM((TAB_CTILES, 8, 128), jnp.float32),
            pltpu.VMEM((OUT_CTILES, 8, 128), jnp.float32),
            pltpu.SemaphoreType.DMA,
        ],
        compiler_params=pltpu.CompilerParams(
            use_tc_tiling_on_sc=False, needs_layout_passes=False
        ),
    )(_gather_body)
    out4 = run(positions.astype(jnp.int32), pe4)
    return out4.transpose(0, 2, 1, 3).reshape(DIM, BATCH).T
